# baseline (speedup 1.0000x reference)
import jax
import jax.numpy as jnp
from jax.experimental import pallas as pl

_B = 128


def _rot_body(seed_ref, sw_ref, c_ref, s_ref, o_ref):
    o_ref[...] = seed_ref[...] * c_ref[0] + sw_ref[...] * s_ref[0]


def kernel(x, pe):
    seq_len = x.shape[-1]
    d = pe.shape[-1]
    nb = seq_len // _B

    seed = pe[:_B]
    sw = jnp.stack([pe[:_B, 1::2], -pe[:_B, 0::2]], axis=-1).reshape(_B, d)
    rows = pe[_B - 1 : (nb - 1) * _B : _B]
    c = jnp.concatenate(
        [jnp.ones((1, d), jnp.float32), jnp.repeat(rows[:, 1::2], 2, axis=1)], axis=0
    )
    s = jnp.concatenate(
        [jnp.zeros((1, d), jnp.float32), jnp.repeat(rows[:, 0::2], 2, axis=1)], axis=0
    )

    c = c.reshape(nb, 1, d)
    s = s.reshape(nb, 1, d)

    out = pl.pallas_call(
        _rot_body,
        grid=(nb,),
        in_specs=[
            pl.BlockSpec((_B, d), lambda i: (0, 0)),
            pl.BlockSpec((_B, d), lambda i: (0, 0)),
            pl.BlockSpec((1, 1, d), lambda i: (i, 0, 0)),
            pl.BlockSpec((1, 1, d), lambda i: (i, 0, 0)),
        ],
        out_specs=pl.BlockSpec((_B, d), lambda i: (i, 0)),
        out_shape=jax.ShapeDtypeStruct((seq_len, d), jnp.float32),
    )(seed, sw, c, s)
    return out[None]
